# baseline (speedup 1.0000x reference)
import functools

import jax
import jax.numpy as jnp
from jax import lax
from jax.experimental import pallas as pl
from jax.experimental.pallas import tpu as pltpu
from jax.experimental.pallas import tpu_sc as plsc

NUM = 16384
C = 1000
C_PAD = 1024

NC = 2
NS = 16
L = 16
NW = NC * NS
BPW = NUM // NW
CHUNK = 128
NCH = BPW // CHUNK
VPW = BPW // L
ROWS_OUT = NUM // CHUNK


def _sc_gather(flat_inputs, targets, alpha_pad):
    mesh = plsc.VectorSubcoreMesh(core_axis_name="c", subcore_axis_name="s")

    @functools.partial(
        pl.kernel,
        mesh=mesh,
        out_type=(
            jax.ShapeDtypeStruct((ROWS_OUT, CHUNK), jnp.float32),
            jax.ShapeDtypeStruct((ROWS_OUT, CHUNK), jnp.float32),
        ),
        scratch_types=[
            pltpu.VMEM((NCH, CHUNK), jnp.int32),
            pltpu.VMEM((NCH, CHUNK), jnp.int32),
            pltpu.VMEM((NCH, CHUNK), jnp.float32),
            pltpu.VMEM((NCH, CHUNK), jnp.float32),
            pltpu.SemaphoreType.DMA,
        ],
    )
    def k(flat_hbm, tgt_hbm, alpha_hbm, p_hbm, a_hbm,
          tgt_v, idx_v, p_v, a_v, sem):
        wid = lax.axis_index("s") * NC + lax.axis_index("c")
        base = wid * BPW
        pltpu.sync_copy(tgt_hbm.at[pl.ds(NCH * wid, NCH)], tgt_v)
        lane = lax.iota(jnp.int32, L)
        for j in range(VPW):
            row0 = base + j * L
            r, c0 = j // (CHUNK // L), (j % (CHUNK // L)) * L
            t_vec = tgt_v[r, pl.ds(c0, L)]
            idx_v[r, pl.ds(c0, L)] = (lane + row0) * C + t_vec
        copies = [
            pltpu.async_copy(flat_hbm.at[idx_v.at[ch]], p_v.at[ch], sem)
            for ch in range(NCH)
        ] + [
            pltpu.async_copy(alpha_hbm.at[tgt_v.at[ch]], a_v.at[ch], sem)
            for ch in range(NCH)
        ]
        for cp in copies:
            cp.wait()
        pltpu.sync_copy(p_v, p_hbm.at[pl.ds(NCH * wid, NCH)])
        pltpu.sync_copy(a_v, a_hbm.at[pl.ds(NCH * wid, NCH)])

    return k(flat_inputs, targets, alpha_pad)


def _tc_body(p_ref, a_ref, o_ref):
    p = p_ref[...]
    a = a_ref[...]
    om = 1.0 - p
    loss = -(a * om * om) * jnp.log(p)
    o_ref[0, 0] = jnp.sum(loss) * (1.0 / NUM)


def _tc_focal(p2d, a2d):
    out = pl.pallas_call(
        _tc_body,
        out_shape=jax.ShapeDtypeStruct((1, 1), jnp.float32),
        out_specs=pl.BlockSpec(memory_space=pltpu.SMEM),
    )(p2d, a2d)
    return out[0, 0]


def kernel(inputs, targets, alpha):
    tgt = targets.astype(jnp.int32).reshape(ROWS_OUT, CHUNK)
    flat = inputs.reshape(-1)
    al = jnp.pad(alpha.reshape(-1), (0, C_PAD - C))
    p2d, a2d = _sc_gather(flat, tgt, al)
    return _tc_focal(p2d, a2d)


# speedup vs baseline: 1.7842x; 1.7842x over previous
import jax
import jax.numpy as jnp
from jax.experimental import pallas as pl
from jax.experimental.pallas import tpu as pltpu

NUM = 16384
C = 1000
ROWS_BLK = 1024
GRID = NUM // ROWS_BLK


def _focal_body(t_ref, al_ref, x_ref, o_ref):
    i = pl.program_id(0)
    x = x_ref[...]
    t = t_ref[0, 0, :]
    cols = jax.lax.broadcasted_iota(jnp.int32, (ROWS_BLK, C), 1)
    mask = (cols == t[:, None]).astype(jnp.float32)
    p = jnp.sum(x * mask, axis=1)
    a = jax.lax.dot_general(mask, al_ref[...], (((1,), (1,)), ((), ())),
                            preferred_element_type=jnp.float32)[:, 0]
    om = 1.0 - p
    part = jnp.sum((a * om * om) * jnp.log(p))

    @pl.when(i == 0)
    def _():
        o_ref[0, 0] = 0.0

    o_ref[0, 0] += part * (-1.0 / NUM)


def kernel(inputs, targets, alpha):
    t3d = targets.astype(jnp.int32).reshape(GRID, 1, ROWS_BLK)
    al2d = alpha.reshape(1, C)
    out = pl.pallas_call(
        _focal_body,
        grid=(GRID,),
        in_specs=[
            pl.BlockSpec((1, 1, ROWS_BLK), lambda i: (i, 0, 0)),
            pl.BlockSpec((1, C), lambda i: (0, 0)),
            pl.BlockSpec((ROWS_BLK, C), lambda i: (i, 0)),
        ],
        out_specs=pl.BlockSpec(memory_space=pltpu.SMEM),
        out_shape=jax.ShapeDtypeStruct((1, 1), jnp.float32),
    )(t3d, al2d, inputs)
    return out[0, 0]


# speedup vs baseline: 1.8661x; 1.0459x over previous
import jax
import jax.numpy as jnp
from jax.experimental import pallas as pl
from jax.experimental.pallas import tpu as pltpu

NUM = 16384
C = 1000
ROWS_BLK = 2048
GRID = NUM // ROWS_BLK


def _focal_body(t_ref, al_ref, x_ref, o_ref):
    i = pl.program_id(0)
    x = x_ref[...]
    t = t_ref[0, 0, :]
    cols = jax.lax.broadcasted_iota(jnp.int32, (ROWS_BLK, C), 1)
    mask = (cols == t[:, None]).astype(jnp.float32)
    p = jnp.sum(x * mask, axis=1)
    a = jax.lax.dot_general(mask, al_ref[...], (((1,), (1,)), ((), ())),
                            preferred_element_type=jnp.float32)[:, 0]
    om = 1.0 - p
    part = jnp.sum((a * om * om) * jnp.log(p))

    @pl.when(i == 0)
    def _():
        o_ref[0, 0] = 0.0

    o_ref[0, 0] += part * (-1.0 / NUM)


def kernel(inputs, targets, alpha):
    t3d = targets.astype(jnp.int32).reshape(GRID, 1, ROWS_BLK)
    al2d = alpha.reshape(1, C)
    out = pl.pallas_call(
        _focal_body,
        grid=(GRID,),
        in_specs=[
            pl.BlockSpec((1, 1, ROWS_BLK), lambda i: (i, 0, 0)),
            pl.BlockSpec((1, C), lambda i: (0, 0)),
            pl.BlockSpec((ROWS_BLK, C), lambda i: (i, 0)),
        ],
        out_specs=pl.BlockSpec(memory_space=pltpu.SMEM),
        out_shape=jax.ShapeDtypeStruct((1, 1), jnp.float32),
    )(t3d, al2d, inputs)
    return out[0, 0]
